# Initial kernel scaffold; baseline (speedup 1.0000x reference)
import jax
import jax.numpy as jnp
from jax.experimental import pallas as pl


def kernel(x_drug, edge_index_rel0, edge_index_rel1, Wl0, Wr0, b0, Wl1, Wr1, b1, M_rel0, R, D_rel1):
    raise NotImplementedError("write your pallas kernel here")


# speedup vs baseline: 2.6673x; 2.6673x over previous
import functools

import jax
import jax.numpy as jnp
from jax import lax
from jax.experimental import pallas as pl
from jax.experimental.pallas import tpu as pltpu
from jax.experimental.pallas import tpu_sc as plsc

N = 10000
E = 320000
D = 128

NC = 2
NS = 16
L = 16


_EB = 80
_EPT_ENC = E // NS
_NCH_ENC = _EPT_ENC // _EB
_NZCH = N // _EB


def _encoder_body(x_hbm, src0, dst0, src1, dst1, znd, ones_hbm, lin,
                  sum0_o, cnt0_o, sum1_o, cnt1_o,
                  accum, rows, sidx, didx, ones_v, lidx, sem):
    c = lax.axis_index("c")
    s = lax.axis_index("s")

    def zero_accum():
        pltpu.sync_copy(znd.at[pl.ds(0, _EB)], rows)
        for m in range(8):
            k = s + 16 * m

            @pl.when(k < _NZCH)
            def _():
                pltpu.sync_copy(lin.at[pl.ds(k * _EB, _EB)], lidx)
                pltpu.async_copy(rows, accum.at[lidx], sem).wait()

    def dump_accum(out_ref):
        for m in range(8):
            k = s + 16 * m

            @pl.when(k < _NZCH)
            def _():
                off = k * _EB
                pltpu.sync_copy(lin.at[pl.ds(off, _EB)], lidx)
                pltpu.async_copy(accum.at[lidx], rows, sem).wait()
                pltpu.sync_copy(rows, out_ref.at[pl.ds(off, _EB)])

    def run(src_hbm, dst_hbm, sum_out, cnt_out):
        zero_accum()
        plsc.subcore_barrier()

        base = s * _EPT_ENC

        def chunk(i, _):
            off = base + i * _EB
            pltpu.sync_copy(src_hbm.at[pl.ds(off, _EB)], sidx)
            pltpu.async_copy(x_hbm.at[sidx], rows, sem).wait()
            pltpu.sync_copy(dst_hbm.at[pl.ds(off, _EB)], didx)
            pltpu.sync_copy(rows, accum.at[didx], add=True)
            return 0

        lax.fori_loop(0, _NCH_ENC, chunk, 0)
        plsc.subcore_barrier()
        dump_accum(sum_out)
        plsc.subcore_barrier()

        zero_accum()
        pltpu.sync_copy(ones_hbm, ones_v)
        plsc.subcore_barrier()

        def chunk2(i, _):
            off = base + i * _EB
            pltpu.sync_copy(dst_hbm.at[pl.ds(off, _EB)], didx)
            pltpu.sync_copy(ones_v, accum.at[didx], add=True)
            return 0

        lax.fori_loop(0, _NCH_ENC, chunk2, 0)
        plsc.subcore_barrier()
        dump_accum(cnt_out)

    @pl.when(c == 0)
    def _():
        run(src0, dst0, sum0_o, cnt0_o)

    @pl.when(c == 1)
    def _():
        run(src1, dst1, sum1_o, cnt1_o)


@jax.jit
def _encode(x, src0, dst0, src1, dst1):
    znd = jnp.zeros((N, D), jnp.float32)
    ones_hbm = jnp.ones((_EB, D), jnp.float32)
    lin = jnp.arange(N, dtype=jnp.int32)
    mesh = plsc.VectorSubcoreMesh(core_axis_name="c", subcore_axis_name="s")
    f = pl.kernel(
        _encoder_body,
        out_type=[
            jax.ShapeDtypeStruct((N, D), jnp.float32),
            jax.ShapeDtypeStruct((N, D), jnp.float32),
            jax.ShapeDtypeStruct((N, D), jnp.float32),
            jax.ShapeDtypeStruct((N, D), jnp.float32),
        ],
        mesh=mesh,
        scratch_types=[
            pltpu.VMEM_SHARED((N, D), jnp.float32),
            pltpu.VMEM((_EB, D), jnp.float32),
            pltpu.VMEM((_EB,), jnp.int32),
            pltpu.VMEM((_EB,), jnp.int32),
            pltpu.VMEM((_EB, D), jnp.float32),
            pltpu.VMEM((_EB,), jnp.int32),
            pltpu.SemaphoreType.DMA,
        ],
    )
    return f(x, src0, dst0, src1, dst1, znd, ones_hbm, lin)


def _dense_body(x_ref, s0_ref, c0_ref, s1_ref, c1_ref,
                wl0_ref, wr0_ref, b0_ref, wl1_ref, wr1_ref, b1_ref,
                m_ref, r_ref, d_ref,
                h_ref, g_ref, q_ref):
    x = x_ref[...]
    c0 = jnp.maximum(c0_ref[...][:, 0:1], 1.0)
    c1 = jnp.maximum(c1_ref[...][:, 0:1], 1.0)
    m0 = s0_ref[...] / c0
    m1 = s1_ref[...] / c1
    hp = jax.lax.Precision.HIGHEST
    h = (jnp.dot(m0, wl0_ref[...], precision=hp)
         + jnp.dot(x, wr0_ref[...], precision=hp)
         + jnp.dot(m1, wl1_ref[...], precision=hp)
         + jnp.dot(x, wr1_ref[...], precision=hp)
         + b0_ref[...] + b1_ref[...])
    h = jnp.maximum(h, 0.0)
    h_ref[...] = h
    g_ref[...] = jnp.dot(h, m_ref[...], precision=hp)
    u = h * d_ref[...]
    q = jnp.sum(jnp.dot(u, r_ref[...], precision=hp) * u, axis=1,
                keepdims=True)
    q_ref[...] = jax.nn.sigmoid(jnp.broadcast_to(q, (q.shape[0], D)))


_BN = 1000


@jax.jit
def _dense(x, sum0, cnt0, sum1, cnt1, Wl0, Wr0, b0, Wl1, Wr1, b1, M_rel0, R, dvec):
    row = pl.BlockSpec((_BN, D), lambda i: (i, 0))
    w = pl.BlockSpec((D, D), lambda i: (0, 0))
    b = pl.BlockSpec((1, D), lambda i: (0, 0))
    return pl.pallas_call(
        _dense_body,
        grid=(N // _BN,),
        in_specs=[row, row, row, row, row, w, w, b, w, w, b, w, w, b],
        out_specs=[row, row, row],
        out_shape=[
            jax.ShapeDtypeStruct((N, D), jnp.float32),
            jax.ShapeDtypeStruct((N, D), jnp.float32),
            jax.ShapeDtypeStruct((N, D), jnp.float32),
        ],
    )(x, sum0, cnt0, sum1, cnt1, Wl0, Wr0, b0, Wl1, Wr1, b1, M_rel0, R, dvec)


_EPT_DEC = E // (NC * NS)
_NCH_DEC = _EPT_DEC // _EB


def _decoder_body(g_hbm, h_hbm, qs_hbm, src0, dst0, src1,
                  out0, out1,
                  grow, hrow, sidx, didx, s1idx, tbuf, o1buf, pbuf,
                  sem, sem2):
    c = lax.axis_index("c")
    s = lax.axis_index("s")
    wid = s * NC + c
    base = wid * _EPT_DEC
    zv = jnp.zeros((L,), jnp.float32)

    pbuf[pl.ds(L, L)] = zv

    def chunk0(i, _):
        off = base + i * _EB
        pltpu.sync_copy(src0.at[pl.ds(off, _EB)], sidx)
        pltpu.sync_copy(dst0.at[pl.ds(off, _EB)], didx)
        pltpu.async_copy(g_hbm.at[sidx], grow, sem).wait()
        pltpu.async_copy(h_hbm.at[didx], hrow, sem2).wait()

        def ebody(e, _):
            acc = jnp.zeros((L,), jnp.float32)
            for cc in range(D // L):
                gv = grow[e, pl.ds(cc * L, L)]
                hv = hrow[e, pl.ds(cc * L, L)]
                acc = acc + gv * hv
            pbuf[pl.ds(0, L)] = acc
            v = acc + pbuf[pl.ds(8, L)]
            pbuf[pl.ds(0, L)] = v
            v = v + pbuf[pl.ds(4, L)]
            pbuf[pl.ds(0, L)] = v
            v = v + pbuf[pl.ds(2, L)]
            pbuf[pl.ds(0, L)] = v
            v = v + pbuf[pl.ds(1, L)]
            tbuf[e, pl.ds(0, L)] = 1.0 / (1.0 + jnp.exp(-v))
            return 0

        lax.fori_loop(0, _EB, ebody, 0)
        pltpu.sync_copy(tbuf, out0.at[pl.ds(off, _EB)])
        return 0

    lax.fori_loop(0, _NCH_DEC, chunk0, 0)

    def chunk1(i, _):
        off = base + i * _EB
        pltpu.sync_copy(src1.at[pl.ds(off, _EB)], s1idx)
        pltpu.async_copy(qs_hbm.at[s1idx], hrow, sem).wait()

        def e1(e, _):
            o1buf[e, pl.ds(0, L)] = hrow[e, pl.ds(0, L)]
            return 0

        lax.fori_loop(0, _EB, e1, 0)
        pltpu.sync_copy(o1buf, out1.at[pl.ds(off, _EB)])
        return 0

    lax.fori_loop(0, _NCH_DEC, chunk1, 0)


@jax.jit
def _decode(G, h, qs, src0, dst0, src1):
    mesh = plsc.VectorSubcoreMesh(core_axis_name="c", subcore_axis_name="s")
    f = pl.kernel(
        _decoder_body,
        out_type=[
            jax.ShapeDtypeStruct((E, L), jnp.float32),
            jax.ShapeDtypeStruct((E, L), jnp.float32),
        ],
        mesh=mesh,
        scratch_types=[
            pltpu.VMEM((_EB, D), jnp.float32),
            pltpu.VMEM((_EB, D), jnp.float32),
            pltpu.VMEM((_EB,), jnp.int32),
            pltpu.VMEM((_EB,), jnp.int32),
            pltpu.VMEM((_EB,), jnp.int32),
            pltpu.VMEM((_EB, L), jnp.float32),
            pltpu.VMEM((_EB, L), jnp.float32),
            pltpu.VMEM((2 * L,), jnp.float32),
            pltpu.SemaphoreType.DMA,
            pltpu.SemaphoreType.DMA,
        ],
    )
    return f(G, h, qs, src0, dst0, src1)


def kernel(x_drug, edge_index_rel0, edge_index_rel1, Wl0, Wr0, b0,
           Wl1, Wr1, b1, M_rel0, R, D_rel1):
    src0 = edge_index_rel0[0]
    dst0 = edge_index_rel0[1]
    src1 = edge_index_rel1[0]
    dst1 = edge_index_rel1[1]
    sum0, cnt0, sum1, cnt1 = _encode(x_drug, src0, dst0, src1, dst1)
    dvec = jnp.reshape(D_rel1, (1, D))
    h, G, qs = _dense(x_drug, sum0, cnt0, sum1, cnt1,
                      Wl0, Wr0, b0.reshape(1, D), Wl1, Wr1, b1.reshape(1, D),
                      M_rel0, R, dvec)
    out0w, out1w = _decode(G, h, qs, src0, dst0, src1)
    return (out0w[:, 0], out1w[:, 0])


# speedup vs baseline: 2.7992x; 1.0495x over previous
import functools

import jax
import jax.numpy as jnp
from jax import lax
from jax.experimental import pallas as pl
from jax.experimental.pallas import tpu as pltpu
from jax.experimental.pallas import tpu_sc as plsc

N = 10000
E = 320000
D = 128

NC = 2
NS = 16
L = 16


_EB = 80
_EPT_ENC = E // NS
_NCH_ENC = _EPT_ENC // _EB
_NZCH = N // _EB


def _encoder_body(x_hbm, src0, dst0, src1, dst1, znd, ones_hbm, lin,
                  sum0_o, cnt0_o, sum1_o, cnt1_o,
                  accum, rows, sidx, didx, ones_v, lidx, sem):
    c = lax.axis_index("c")
    s = lax.axis_index("s")

    def zero_accum():
        pltpu.sync_copy(znd.at[pl.ds(0, _EB)], rows)
        for m in range(8):
            k = s + 16 * m

            @pl.when(k < _NZCH)
            def _():
                pltpu.sync_copy(lin.at[pl.ds(k * _EB, _EB)], lidx)
                pltpu.async_copy(rows, accum.at[lidx], sem).wait()

    def dump_accum(out_ref):
        for m in range(8):
            k = s + 16 * m

            @pl.when(k < _NZCH)
            def _():
                off = k * _EB
                pltpu.sync_copy(lin.at[pl.ds(off, _EB)], lidx)
                pltpu.async_copy(accum.at[lidx], rows, sem).wait()
                pltpu.sync_copy(rows, out_ref.at[pl.ds(off, _EB)])

    def run(src_hbm, dst_hbm, sum_out, cnt_out):
        zero_accum()
        plsc.subcore_barrier()

        base = s * _EPT_ENC

        def chunk(i, _):
            off = base + i * _EB
            pltpu.sync_copy(src_hbm.at[pl.ds(off, _EB)], sidx)
            pltpu.async_copy(x_hbm.at[sidx], rows, sem).wait()
            pltpu.sync_copy(dst_hbm.at[pl.ds(off, _EB)], didx)
            pltpu.sync_copy(rows, accum.at[didx], add=True)
            return 0

        lax.fori_loop(0, _NCH_ENC, chunk, 0)
        plsc.subcore_barrier()
        dump_accum(sum_out)
        plsc.subcore_barrier()

        zero_accum()
        pltpu.sync_copy(ones_hbm, ones_v)
        plsc.subcore_barrier()

        def chunk2(i, _):
            off = base + i * _EB
            pltpu.sync_copy(dst_hbm.at[pl.ds(off, _EB)], didx)
            pltpu.sync_copy(ones_v, accum.at[didx], add=True)
            return 0

        lax.fori_loop(0, _NCH_ENC, chunk2, 0)
        plsc.subcore_barrier()
        dump_accum(cnt_out)

    @pl.when(c == 0)
    def _():
        run(src0, dst0, sum0_o, cnt0_o)

    @pl.when(c == 1)
    def _():
        run(src1, dst1, sum1_o, cnt1_o)


@jax.jit
def _encode(x, src0, dst0, src1, dst1):
    znd = jnp.zeros((N, D), jnp.float32)
    ones_hbm = jnp.ones((_EB, D), jnp.float32)
    lin = jnp.arange(N, dtype=jnp.int32)
    mesh = plsc.VectorSubcoreMesh(core_axis_name="c", subcore_axis_name="s")
    f = pl.kernel(
        _encoder_body,
        out_type=[
            jax.ShapeDtypeStruct((N, D), jnp.float32),
            jax.ShapeDtypeStruct((N, D), jnp.float32),
            jax.ShapeDtypeStruct((N, D), jnp.float32),
            jax.ShapeDtypeStruct((N, D), jnp.float32),
        ],
        mesh=mesh,
        scratch_types=[
            pltpu.VMEM_SHARED((N, D), jnp.float32),
            pltpu.VMEM((_EB, D), jnp.float32),
            pltpu.VMEM((_EB,), jnp.int32),
            pltpu.VMEM((_EB,), jnp.int32),
            pltpu.VMEM((_EB, D), jnp.float32),
            pltpu.VMEM((_EB,), jnp.int32),
            pltpu.SemaphoreType.DMA,
        ],
    )
    return f(x, src0, dst0, src1, dst1, znd, ones_hbm, lin)


def _dense_body(x_ref, s0_ref, c0_ref, s1_ref, c1_ref,
                wl0_ref, wr0_ref, b0_ref, wl1_ref, wr1_ref, b1_ref,
                m_ref, r_ref, d_ref,
                h_ref, g_ref, q_ref):
    x = x_ref[...]
    c0 = jnp.maximum(c0_ref[...][:, 0:1], 1.0)
    c1 = jnp.maximum(c1_ref[...][:, 0:1], 1.0)
    m0 = s0_ref[...] / c0
    m1 = s1_ref[...] / c1
    hp = jax.lax.Precision.HIGHEST
    h = (jnp.dot(m0, wl0_ref[...], precision=hp)
         + jnp.dot(x, wr0_ref[...], precision=hp)
         + jnp.dot(m1, wl1_ref[...], precision=hp)
         + jnp.dot(x, wr1_ref[...], precision=hp)
         + b0_ref[...] + b1_ref[...])
    h = jnp.maximum(h, 0.0)
    h_ref[...] = h
    g_ref[...] = jnp.dot(h, m_ref[...], precision=hp)
    u = h * d_ref[...]
    q = jnp.sum(jnp.dot(u, r_ref[...], precision=hp) * u, axis=1,
                keepdims=True)
    q_ref[...] = jax.nn.sigmoid(jnp.broadcast_to(q, (q.shape[0], D)))


_BN = 1000


@jax.jit
def _dense(x, sum0, cnt0, sum1, cnt1, Wl0, Wr0, b0, Wl1, Wr1, b1, M_rel0, R, dvec):
    row = pl.BlockSpec((_BN, D), lambda i: (i, 0))
    w = pl.BlockSpec((D, D), lambda i: (0, 0))
    b = pl.BlockSpec((1, D), lambda i: (0, 0))
    return pl.pallas_call(
        _dense_body,
        grid=(N // _BN,),
        in_specs=[row, row, row, row, row, w, w, b, w, w, b, w, w, b],
        out_specs=[row, row, row],
        out_shape=[
            jax.ShapeDtypeStruct((N, D), jnp.float32),
            jax.ShapeDtypeStruct((N, D), jnp.float32),
            jax.ShapeDtypeStruct((N, D), jnp.float32),
        ],
    )(x, sum0, cnt0, sum1, cnt1, Wl0, Wr0, b0, Wl1, Wr1, b1, M_rel0, R, dvec)


_EPT_DEC = E // (NC * NS)
_NCH_DEC = _EPT_DEC // _EB


def _decoder_body(g_hbm, h_hbm, qs_hbm, src0, dst0, src1,
                  out0, out1,
                  grow, hrow, sidx, didx, s1idx, tbuf, o1buf, pbuf,
                  sem, sem2):
    c = lax.axis_index("c")
    s = lax.axis_index("s")
    wid = s * NC + c
    base = wid * _EPT_DEC
    zv = jnp.zeros((L,), jnp.float32)

    pbuf[pl.ds(L, L)] = zv

    def chunk0(i, _):
        off = base + i * _EB
        pltpu.sync_copy(src0.at[pl.ds(off, _EB)], sidx)
        pltpu.sync_copy(dst0.at[pl.ds(off, _EB)], didx)
        cp1 = pltpu.async_copy(g_hbm.at[sidx], grow, sem)
        cp2 = pltpu.async_copy(h_hbm.at[didx], hrow, sem2)
        cp1.wait()
        cp2.wait()

        def ebody(e, _):
            acc = jnp.zeros((L,), jnp.float32)
            for cc in range(D // L):
                gv = grow[e, pl.ds(cc * L, L)]
                hv = hrow[e, pl.ds(cc * L, L)]
                acc = acc + gv * hv
            pbuf[pl.ds(0, L)] = acc
            v = acc + pbuf[pl.ds(8, L)]
            pbuf[pl.ds(0, L)] = v
            v = v + pbuf[pl.ds(4, L)]
            pbuf[pl.ds(0, L)] = v
            v = v + pbuf[pl.ds(2, L)]
            pbuf[pl.ds(0, L)] = v
            v = v + pbuf[pl.ds(1, L)]
            tbuf[e, pl.ds(0, L)] = 1.0 / (1.0 + jnp.exp(-v))
            return 0

        lax.fori_loop(0, _EB, ebody, 0)
        pltpu.sync_copy(tbuf, out0.at[pl.ds(off, _EB)])
        return 0

    lax.fori_loop(0, _NCH_DEC, chunk0, 0)

    def chunk1(i, _):
        off = base + i * _EB
        pltpu.sync_copy(src1.at[pl.ds(off, _EB)], s1idx)
        pltpu.async_copy(qs_hbm.at[s1idx], hrow, sem).wait()

        def e1(e, _):
            o1buf[e, pl.ds(0, L)] = hrow[e, pl.ds(0, L)]
            return 0

        lax.fori_loop(0, _EB, e1, 0)
        pltpu.sync_copy(o1buf, out1.at[pl.ds(off, _EB)])
        return 0

    lax.fori_loop(0, _NCH_DEC, chunk1, 0)


@jax.jit
def _decode(G, h, qs, src0, dst0, src1):
    mesh = plsc.VectorSubcoreMesh(core_axis_name="c", subcore_axis_name="s")
    f = pl.kernel(
        _decoder_body,
        out_type=[
            jax.ShapeDtypeStruct((E, L), jnp.float32),
            jax.ShapeDtypeStruct((E, L), jnp.float32),
        ],
        mesh=mesh,
        scratch_types=[
            pltpu.VMEM((_EB, D), jnp.float32),
            pltpu.VMEM((_EB, D), jnp.float32),
            pltpu.VMEM((_EB,), jnp.int32),
            pltpu.VMEM((_EB,), jnp.int32),
            pltpu.VMEM((_EB,), jnp.int32),
            pltpu.VMEM((_EB, L), jnp.float32),
            pltpu.VMEM((_EB, L), jnp.float32),
            pltpu.VMEM((2 * L,), jnp.float32),
            pltpu.SemaphoreType.DMA,
            pltpu.SemaphoreType.DMA,
        ],
    )
    return f(G, h, qs, src0, dst0, src1)


def kernel(x_drug, edge_index_rel0, edge_index_rel1, Wl0, Wr0, b0,
           Wl1, Wr1, b1, M_rel0, R, D_rel1):
    src0 = edge_index_rel0[0]
    dst0 = edge_index_rel0[1]
    src1 = edge_index_rel1[0]
    dst1 = edge_index_rel1[1]
    sum0, cnt0, sum1, cnt1 = _encode(x_drug, src0, dst0, src1, dst1)
    dvec = jnp.reshape(D_rel1, (1, D))
    h, G, qs = _dense(x_drug, sum0, cnt0, sum1, cnt1,
                      Wl0, Wr0, b0.reshape(1, D), Wl1, Wr1, b1.reshape(1, D),
                      M_rel0, R, dvec)
    out0w, out1w = _decode(G, h, qs, src0, dst0, src1)
    return (out0w[:, 0], out1w[:, 0])
